# baseline (speedup 1.0000x reference)
import functools

import jax
import jax.numpy as jnp
from jax import lax
from jax.experimental import pallas as pl
from jax.experimental.pallas import tpu as pltpu
from jax.experimental.pallas import tpu_sc as plsc

_NC = 2
_NS = 16
_NW = _NC * _NS


@functools.lru_cache(maxsize=None)
def _make_gather(n_tok: int, d: int):
    tpw = n_tok // _NW

    mesh = plsc.VectorSubcoreMesh(core_axis_name="c", subcore_axis_name="s")

    @functools.partial(
        pl.kernel,
        mesh=mesh,
        out_type=jax.ShapeDtypeStruct((d, n_tok), jnp.float32),
        scratch_types=[
            pltpu.VMEM((tpw,), jnp.int32),
            pltpu.VMEM((tpw,), jnp.float32),
            pltpu.VMEM((tpw,), jnp.float32),
            pltpu.SemaphoreType.DMA,
            pltpu.SemaphoreType.DMA,
            pltpu.SemaphoreType.DMA,
            pltpu.SemaphoreType.DMA,
        ],
        compiler_params=pltpu.CompilerParams(use_tc_tiling_on_sc=False),
    )
    def gather_kernel(tab_t, idx_hbm, out_t, idx_v, v0, v1, g0, g1, o0, o1):
        wid = lax.axis_index("s") * _NC + lax.axis_index("c")
        base = wid * tpw
        bufs = (v0, v1)
        gsems = (g0, g1)
        osems = (o0, o1)

        pltpu.sync_copy(idx_hbm.at[wid], idx_v)

        def fire(e):
            b = e % 2
            return pltpu.async_copy(
                tab_t.at[e].at[idx_v], bufs[b], gsems[b]
            )

        fire(0)
        for e in range(d):
            b = e % 2
            pltpu.make_async_copy(
                tab_t.at[e].at[idx_v], bufs[b], gsems[b]
            ).wait()
            if e + 1 < d:
                fire(e + 1)
            if e >= 2:
                pltpu.make_async_copy(
                    bufs[b], out_t.at[e - 2, pl.ds(base, tpw)], osems[b]
                ).wait()
            pltpu.async_copy(
                bufs[b], out_t.at[e, pl.ds(base, tpw)], osems[b]
            )
        for e in (d - 2, d - 1):
            pltpu.make_async_copy(
                bufs[e % 2], out_t.at[e, pl.ds(base, tpw)], osems[e % 2]
            ).wait()

    return gather_kernel


def kernel(table, input_ids):
    b, h = input_ids.shape
    vocab, d = table.shape
    n = b * h
    idx = input_ids.reshape(_NW, n // _NW).astype(jnp.int32)
    out_t = _make_gather(n, d)(table.T, idx)
    return out_t.T.reshape(b, h, d)


# speedup vs baseline: 7.5767x; 7.5767x over previous
import functools

import jax
import jax.numpy as jnp
from jax import lax
from jax.experimental import pallas as pl
from jax.experimental.pallas import tpu as pltpu
from jax.experimental.pallas import tpu_sc as plsc

_G = 100
_NC = 2
_NS = 16
_NW = _NC * _NS


@functools.lru_cache(maxsize=None)
def _make_gather(b: int, h: int, d: int):
    n = b * h
    tpw = n // _NW
    gpw = tpw // _G
    bpw = b // _NW
    gpc = 8
    rows_pc = gpc * _G
    bpc = rows_pc // h
    n_chunks = gpw // gpc
    assert rows_pc % h == 0 and n_chunks % 2 == 0

    mesh = plsc.VectorSubcoreMesh(core_axis_name="c", subcore_axis_name="s")

    @functools.partial(
        pl.kernel,
        mesh=mesh,
        out_type=jax.ShapeDtypeStruct((b, h, d), jnp.float32),
        scratch_types=[
            pltpu.VMEM((gpw, _G), jnp.int32),
            pltpu.VMEM((rows_pc, d), jnp.float32),
            pltpu.VMEM((rows_pc, d), jnp.float32),
            pltpu.SemaphoreType.DMA,
            pltpu.SemaphoreType.DMA,
            pltpu.SemaphoreType.DMA,
            pltpu.SemaphoreType.DMA,
        ],
        compiler_params=pltpu.CompilerParams(use_tc_tiling_on_sc=False),
    )
    def gather_kernel(table_hbm, idx_hbm, out_hbm, idx_v,
                      rows0, rows1, g0, g1, o0, o1):
        wid = lax.axis_index("s") * _NC + lax.axis_index("c")
        bufs = (rows0, rows1)
        gsems = (g0, g1)
        osems = (o0, o1)

        pltpu.sync_copy(idx_hbm.at[wid], idx_v)

        def gathers(ci, bb):
            return [
                pltpu.make_async_copy(
                    table_hbm.at[idx_v.at[ci * gpc + g]],
                    bufs[bb].at[pl.ds(g * _G, _G)],
                    gsems[bb],
                )
                for g in range(gpc)
            ]

        def writes(ci, bb):
            return [
                pltpu.make_async_copy(
                    bufs[bb].at[pl.ds(k * h, h)],
                    out_hbm.at[wid * bpw + ci * bpc + k],
                    osems[bb],
                )
                for k in range(bpc)
            ]

        def fire(copies):
            for c in copies:
                c.start()

        def wait(copies):
            for c in copies:
                c.wait()

        fire(gathers(0, 0))

        def body(j, carry):
            c0 = 2 * j
            c1 = c0 + 1
            wait(gathers(c0, 0))

            @pl.when(j > 0)
            def _():
                wait(writes(c1 - 2, 1))

            fire(gathers(c1, 1))
            fire(writes(c0, 0))
            wait(gathers(c1, 1))
            wait(writes(c0, 0))

            @pl.when(j + 1 < n_chunks // 2)
            def _():
                fire(gathers(c0 + 2, 0))

            fire(writes(c1, 1))
            return carry

        lax.fori_loop(0, n_chunks // 2, body, 0)
        wait(writes(n_chunks - 1, 1))

    return gather_kernel


def kernel(table, input_ids):
    b, h = input_ids.shape
    d = table.shape[1]
    idx = input_ids.reshape(_NW, (b * h) // (_NW * _G), _G).astype(jnp.int32)
    return _make_gather(b, h, d)(table, idx)
